# baseline (speedup 1.0000x reference)
import functools

import jax
import jax.numpy as jnp
from jax.experimental import pallas as pl
from jax.experimental.pallas import tpu as pltpu

_SPLITS = 2
_BLOCK_M = 200


def _gcn_body(*refs, n_steps, block_m, n_rows, splits):
    (x_ref, *adj_refs, w1_ref, b1_ref, w2_ref, b2_ref, wr_ref, br_ref,
     out_ref, y1_ref, y2_ref, acc_ref) = refs
    p = pl.program_id(0)
    i = pl.program_id(1)

    @pl.when((p == 0) & (i == 0))
    def _init():
        y1_ref[...] = jnp.dot(x_ref[...], w1_ref[...],
                              precision=jax.lax.Precision.DEFAULT,
                              preferred_element_type=jnp.float32)
        acc_ref[...] = jnp.zeros_like(acc_ref)

    @pl.when(p == 0)
    def _layer1():
        for s in range(splits):
            t = jnp.dot(adj_refs[s][...], y1_ref[...],
                        precision=jax.lax.Precision.DEFAULT,
                        preferred_element_type=jnp.float32)
            hs = jnp.maximum(t + b1_ref[...], 0.0)
            y2_ref[pl.ds((i * splits + s) * block_m, block_m), :] = jnp.dot(
                hs, w2_ref[...],
                precision=jax.lax.Precision.DEFAULT,
                preferred_element_type=jnp.float32)

    @pl.when(p == 1)
    def _layer2():
        part = jnp.zeros_like(acc_ref)
        for s in range(splits):
            t = jnp.dot(adj_refs[s][...], y2_ref[...],
                        precision=jax.lax.Precision.DEFAULT,
                        preferred_element_type=jnp.float32)
            r = jnp.maximum(t + b2_ref[...], 0.0)
            part = part + jnp.sum(r, axis=0, keepdims=True)
        acc_ref[...] += part

    @pl.when((p == 1) & (i == n_steps - 1))
    def _readout():
        g = acc_ref[...] * (1.0 / n_rows)
        out_ref[...] = jnp.dot(g, wr_ref[...],
                               precision=jax.lax.Precision.DEFAULT,
                               preferred_element_type=jnp.float32) + br_ref[...]


def kernel(x, adj, W1, b1, W2, b2, Wr, br):
    n, f = x.shape
    h = W1.shape[1]
    op = Wr.shape[1]
    splits, block_m = _SPLITS, _BLOCK_M
    if n % (splits * block_m) != 0:
        splits, block_m = 1, 400 if n % 400 == 0 else 8
    n_steps = n // (splits * block_m)

    def _adj_spec(s):
        return pl.BlockSpec((block_m, n),
                            lambda p, i, s=s: (i * splits + s, 0))

    out = pl.pallas_call(
        functools.partial(_gcn_body, n_steps=n_steps, block_m=block_m,
                          n_rows=n, splits=splits),
        grid=(2, n_steps),
        in_specs=(
            [pl.BlockSpec((n, f), lambda p, i: (0, 0))]
            + [_adj_spec(s) for s in range(splits)]
            + [
                pl.BlockSpec((f, h), lambda p, i: (0, 0)),
                pl.BlockSpec((1, h), lambda p, i: (0, 0)),
                pl.BlockSpec((h, h), lambda p, i: (0, 0)),
                pl.BlockSpec((1, h), lambda p, i: (0, 0)),
                pl.BlockSpec((h, op), lambda p, i: (0, 0)),
                pl.BlockSpec((1, op), lambda p, i: (0, 0)),
            ]
        ),
        out_specs=pl.BlockSpec((1, op), lambda p, i: (0, 0)),
        out_shape=jax.ShapeDtypeStruct((1, op), jnp.float32),
        scratch_shapes=[
            pltpu.VMEM((n, h), jnp.float32),
            pltpu.VMEM((n, h), jnp.float32),
            pltpu.VMEM((1, h), jnp.float32),
        ],
    )(x, *([adj] * splits), W1, b1.reshape(1, h), W2, b2.reshape(1, h), Wr,
      br.reshape(1, op))
    return out.reshape(op // 4, 4)
